# baseline (speedup 1.0000x reference)
import jax
import jax.numpy as jnp
from jax.experimental import pallas as pl
from jax.experimental.pallas import tpu as pltpu


def _fts_kernel(seq_ref, wt_ref, fts_ref):
    fts = jnp.dot(seq_ref[...], wt_ref[...],
                  preferred_element_type=jnp.float32,
                  precision=jax.lax.Precision.HIGHEST)
    fts_ref[...] = fts.astype(jnp.bfloat16)


def _agg_kernel(fts_ref, b_ref, adj_ref, out_ref):
    acc = jnp.dot(adj_ref[...].astype(jnp.bfloat16), fts_ref[...],
                  preferred_element_type=jnp.float32)
    out_ref[...] = acc + b_ref[...]


def kernel(seq, adj, W, b):
    batch, n, in_ft = seq.shape
    out_ft = W.shape[0]
    seq2 = seq.reshape(batch * n, in_ft)
    adj2 = adj.reshape(batch * n, n)
    wt = W.T
    b2 = b.reshape(1, out_ft)

    fm = 2000
    fts = pl.pallas_call(
        _fts_kernel,
        grid=(n // fm,),
        in_specs=[
            pl.BlockSpec((fm, in_ft), lambda i: (i, 0)),
            pl.BlockSpec((in_ft, out_ft), lambda i: (0, 0)),
        ],
        out_specs=pl.BlockSpec((fm, out_ft), lambda i: (i, 0)),
        out_shape=jax.ShapeDtypeStruct((n, out_ft), jnp.bfloat16),
        compiler_params=pltpu.CompilerParams(
            dimension_semantics=("arbitrary",),
        ),
    )(seq2, wt)

    bm = 400
    out = pl.pallas_call(
        _agg_kernel,
        grid=(n // bm,),
        in_specs=[
            pl.BlockSpec((n, out_ft), lambda i: (0, 0)),
            pl.BlockSpec((1, out_ft), lambda i: (0, 0)),
            pl.BlockSpec((bm, n), lambda i: (i, 0)),
        ],
        out_specs=pl.BlockSpec((bm, out_ft), lambda i: (i, 0)),
        out_shape=jax.ShapeDtypeStruct((n, out_ft), jnp.float32),
        compiler_params=pltpu.CompilerParams(
            dimension_semantics=("arbitrary",),
        ),
    )(fts, b2, adj2)

    return out.reshape(batch, n, out_ft)
